# Initial kernel scaffold; baseline (speedup 1.0000x reference)
import jax
import jax.numpy as jnp
from jax.experimental import pallas as pl


def kernel(feature_maps, sample_points):
    raise NotImplementedError("write your pallas kernel here")


# speedup vs baseline: 6.2781x; 6.2781x over previous
import functools

import jax
import jax.numpy as jnp
from jax import lax
from jax.experimental import pallas as pl
from jax.experimental.pallas import tpu as pltpu
from jax.experimental.pallas import tpu_sc as plsc

B, N, D, H, W = 8, 4096, 384, 32, 32
NPTS = B * N

_info = plsc.get_sparse_core_info()
NC, NS, L = _info.num_cores, _info.num_subcores, _info.num_lanes
NW = NC * NS
PPW = NPTS // NW
G = 16
NG = PPW // G


def _sc_sample(table, xs, ys):
  mesh = plsc.VectorSubcoreMesh(core_axis_name="c", subcore_axis_name="s")

  @functools.partial(
      pl.kernel,
      mesh=mesh,
      out_type=jax.ShapeDtypeStruct((NPTS, D), jnp.float32),
      scratch_types=[
          pltpu.VMEM((PPW,), jnp.float32),
          pltpu.VMEM((PPW,), jnp.float32),
          pltpu.VMEM((4 * G,), jnp.int32),
          pltpu.VMEM((4 * G, D), jnp.float32),
          pltpu.VMEM((G, D), jnp.float32),
          pltpu.SemaphoreType.DMA,
      ],
  )
  def k(table_hbm, xs_hbm, ys_hbm, out_hbm,
        xs_v, ys_v, idx_v, rows_v, out_v, sem):
    wid = lax.axis_index("s") * NC + lax.axis_index("c")
    base = wid * PPW
    bbase = (base // N) * (H * W)
    pltpu.sync_copy(xs_hbm.at[pl.ds(base, PPW)], xs_v)
    pltpu.sync_copy(ys_hbm.at[pl.ds(base, PPW)], ys_v)

    def group(g, carry):
      off = g * G
      x = xs_v[pl.ds(off, L)]
      y = ys_v[pl.ds(off, L)]
      gx = x * 2.0 - 1.0
      gy = y * 2.0 - 1.0
      ix = ((gx + 1.0) * W - 1.0) * 0.5
      iy = ((gy + 1.0) * H - 1.0) * 0.5
      xt = ix.astype(jnp.int32)
      x0 = jnp.where(ix < xt.astype(jnp.float32), xt - 1, xt)
      yt = iy.astype(jnp.int32)
      y0 = jnp.where(iy < yt.astype(jnp.float32), yt - 1, yt)
      wx1 = ix - x0.astype(jnp.float32)
      wy1 = iy - y0.astype(jnp.float32)
      wx0 = 1.0 - wx1
      wy0 = 1.0 - wy1
      x1 = x0 + 1
      y1 = y0 + 1
      wx0 = jnp.where((x0 >= 0) & (x0 < W), wx0, 0.0)
      wx1 = jnp.where((x1 >= 0) & (x1 < W), wx1, 0.0)
      wy0 = jnp.where((y0 >= 0) & (y0 < H), wy0, 0.0)
      wy1 = jnp.where((y1 >= 0) & (y1 < H), wy1, 0.0)
      x0c = jnp.clip(x0, 0, W - 1)
      x1c = jnp.clip(x1, 0, W - 1)
      y0c = jnp.clip(y0, 0, H - 1)
      y1c = jnp.clip(y1, 0, H - 1)
      r0 = bbase + y0c * W
      r1 = bbase + y1c * W
      idx_v[pl.ds(0 * L, L)] = r0 + x0c
      idx_v[pl.ds(1 * L, L)] = r0 + x1c
      idx_v[pl.ds(2 * L, L)] = r1 + x0c
      idx_v[pl.ds(3 * L, L)] = r1 + x1c
      w = [wy0 * wx0, wy0 * wx1, wy1 * wx0, wy1 * wx1]
      pltpu.async_copy(table_hbm.at[idx_v], rows_v, sem).wait()
      for i in range(G):
        lane = jnp.full((L,), i, jnp.int32)
        wb = [w[c].at[lane].get(mode="promise_in_bounds") for c in range(4)]
        for kk in range(D // L):
          sl = pl.ds(kk * L, L)
          acc = wb[0] * rows_v[0 * G + i, sl]
          acc = acc + wb[1] * rows_v[1 * G + i, sl]
          acc = acc + wb[2] * rows_v[2 * G + i, sl]
          acc = acc + wb[3] * rows_v[3 * G + i, sl]
          out_v[i, sl] = acc
      pltpu.sync_copy(out_v, out_hbm.at[pl.ds(base + off, G), :])
      return carry

    lax.fori_loop(0, NG, group, 0)

  return k(table, xs, ys)


def kernel(feature_maps, sample_points):
  fm_t = jnp.transpose(feature_maps, (0, 2, 3, 1)).reshape(B * H * W, D)
  pts = sample_points.reshape(NPTS, 2)
  out = _sc_sample(fm_t, pts[:, 0], pts[:, 1])
  return out.reshape(B, N, D)
